# Initial kernel scaffold; baseline (speedup 1.0000x reference)
import jax
import jax.numpy as jnp
from jax.experimental import pallas as pl


def kernel(keyframe_attns, w, before_idx, after_idx, corr_before, corr_after):
    raise NotImplementedError("write your pallas kernel here")


# speedup vs baseline: 1.0318x; 1.0318x over previous
import functools

import jax
import jax.numpy as jnp
from jax import lax
from jax.experimental import pallas as pl
from jax.experimental.pallas import tpu as pltpu
from jax.experimental.pallas import tpu_sc as plsc

LANES = 16


def _make_sc_kernel(KS, D, BL, B, L, nw, interpret=False):
    rpw = BL // nw
    R = 16
    n_chunks = rpw // R

    mesh = plsc.VectorSubcoreMesh(core_axis_name="c", subcore_axis_name="s")
    NC = mesh.num_cores

    @functools.partial(
        pl.kernel,
        out_type=jax.ShapeDtypeStruct((BL, D), jnp.float32),
        mesh=mesh,
        scratch_types=[
            pltpu.VMEM((rpw,), jnp.int32),
            pltpu.VMEM((rpw,), jnp.int32),
            pltpu.VMEM((LANES,), jnp.int32),
            pltpu.VMEM((LANES,), jnp.int32),
            pltpu.VMEM((LANES,), jnp.float32),
            pltpu.VMEM((R, D), jnp.float32),
            pltpu.VMEM((R, D), jnp.float32),
            pltpu.SemaphoreType.DMA,
        ],
        interpret=interpret,
    )
    def sc_kernel(table_hbm, corr_b_hbm, corr_a_hbm, base_b_hbm, base_a_hbm,
                  w_hbm, out_hbm, idxb_v, idxa_v, bb_v, ba_v, w_v,
                  rows_b, rows_a, sem):
        wid = lax.axis_index("s") * NC + lax.axis_index("c")
        row0 = wid * rpw
        b = row0 // L

        pltpu.sync_copy(corr_b_hbm.at[pl.ds(row0, rpw)], idxb_v)
        pltpu.sync_copy(corr_a_hbm.at[pl.ds(row0, rpw)], idxa_v)
        pltpu.sync_copy(base_b_hbm.at[b], bb_v)
        pltpu.sync_copy(base_a_hbm.at[b], ba_v)
        pltpu.sync_copy(w_hbm.at[b], w_v)
        bb = bb_v[...]
        ba = ba_v[...]
        wv = w_v[...]

        def chunk(g, carry):
            idx_b = idxb_v[pl.ds(g * R, R)] + bb
            idx_a = idxa_v[pl.ds(g * R, R)] + ba
            cp_b = pltpu.async_copy(table_hbm.at[idx_b], rows_b, sem)
            cp_a = pltpu.async_copy(table_hbm.at[idx_a], rows_a, sem)
            cp_b.wait()
            cp_a.wait()

            def blend_row(r, c2):
                def blend_col(c, c3):
                    xb = rows_b[r, pl.ds(c * LANES, LANES)]
                    xa = rows_a[r, pl.ds(c * LANES, LANES)]
                    rows_b[r, pl.ds(c * LANES, LANES)] = xb + wv * (xa - xb)
                    return c3
                return lax.fori_loop(0, D // LANES, blend_col, c2, unroll=8)

            lax.fori_loop(0, R, blend_row, 0)
            pltpu.sync_copy(rows_b, out_hbm.at[pl.ds(row0 + g * R, R)])
            return carry

        lax.fori_loop(0, n_chunks, chunk, 0)

    return sc_kernel


def kernel(keyframe_attns, w, before_idx, after_idx, corr_before, corr_after):
    K, S, D = keyframe_attns.shape
    B, L = corr_before.shape
    nw = 32
    table = keyframe_attns.reshape(K * S, D)
    corr_b = corr_before.reshape(B * L).astype(jnp.int32)
    corr_a = corr_after.reshape(B * L).astype(jnp.int32)
    base_b = jnp.broadcast_to((before_idx.astype(jnp.int32) * S)[:, None], (B, LANES))
    base_a = jnp.broadcast_to((after_idx.astype(jnp.int32) * S)[:, None], (B, LANES))
    w_rep = jnp.broadcast_to(w.astype(jnp.float32)[:, None], (B, LANES))
    sc = _make_sc_kernel(K * S, D, B * L, B, L, nw)
    out = sc(table, corr_b, corr_a, base_b, base_a, w_rep)
    return out.reshape(B, L, D)


# speedup vs baseline: 3.9150x; 3.7943x over previous
import functools

import jax
import jax.numpy as jnp
from jax import lax
from jax.experimental import pallas as pl
from jax.experimental.pallas import tpu as pltpu
from jax.experimental.pallas import tpu_sc as plsc

LANES = 16


def _make_sc_kernel(KS, D, BL, B, L, nw, interpret=False):
    rpw = BL // nw
    R = 16
    n_chunks = rpw // R

    mesh = plsc.VectorSubcoreMesh(core_axis_name="c", subcore_axis_name="s")
    NC = mesh.num_cores

    @functools.partial(
        pl.kernel,
        out_type=jax.ShapeDtypeStruct((BL, D), jnp.float32),
        mesh=mesh,
        scratch_types=[
            pltpu.VMEM((rpw,), jnp.int32),
            pltpu.VMEM((rpw,), jnp.int32),
            pltpu.VMEM((LANES,), jnp.int32),
            pltpu.VMEM((LANES,), jnp.int32),
            pltpu.VMEM((LANES,), jnp.float32),
            pltpu.VMEM((R, D), jnp.float32),
            pltpu.VMEM((R, D), jnp.float32),
            pltpu.VMEM((R, D), jnp.float32),
            pltpu.VMEM((R, D), jnp.float32),
            pltpu.VMEM((R, D), jnp.float32),
            pltpu.VMEM((R, D), jnp.float32),
            pltpu.SemaphoreType.DMA,
            pltpu.SemaphoreType.DMA,
            pltpu.SemaphoreType.DMA,
            pltpu.SemaphoreType.DMA,
        ],
        interpret=interpret,
    )
    def sc_kernel(table_hbm, corr_b_hbm, corr_a_hbm, base_b_hbm, base_a_hbm,
                  w_hbm, out_hbm, idxb_v, idxa_v, bb_v, ba_v, w_v,
                  gb0, gb1, ga0, ga1, ob0, ob1, gsem0, gsem1, osem0, osem1):
        wid = lax.axis_index("s") * NC + lax.axis_index("c")
        row0 = wid * rpw
        b = row0 // L

        pltpu.sync_copy(corr_b_hbm.at[pl.ds(row0, rpw)], idxb_v)
        pltpu.sync_copy(corr_a_hbm.at[pl.ds(row0, rpw)], idxa_v)
        pltpu.sync_copy(base_b_hbm.at[b], bb_v)
        pltpu.sync_copy(base_a_hbm.at[b], ba_v)
        pltpu.sync_copy(w_hbm.at[b], w_v)
        bb = bb_v[...]
        ba = ba_v[...]
        wv = w_v[...]

        gbufs = ((gb0, ga0, gsem0), (gb1, ga1, gsem1))
        obufs = ((ob0, osem0), (ob1, osem1))

        def gather_descs(g, par):
            gbuf_b, gbuf_a, gsem = gbufs[par]
            idx_b = idxb_v[pl.ds(g * R, R)] + bb
            idx_a = idxa_v[pl.ds(g * R, R)] + ba
            db = pltpu.make_async_copy(table_hbm.at[idx_b], gbuf_b, gsem)
            da = pltpu.make_async_copy(table_hbm.at[idx_a], gbuf_a, gsem)
            return db, da

        d0b, d0a = gather_descs(0, 0)
        d0b.start()
        d0a.start()

        def pair(go, carry):
            for par in range(2):
                g = go * 2 + par
                gbuf_b, gbuf_a, _ = gbufs[par]
                obuf, osem = obufs[par]

                @pl.when(g + 1 < n_chunks)
                def _issue_next():
                    dnb, dna = gather_descs(g + 1, 1 - par)
                    dnb.start()
                    dna.start()

                dwb, dwa = gather_descs(g, par)
                dwb.wait()
                dwa.wait()

                @pl.when(g >= 2)
                def _drain_prev_write():
                    pltpu.make_async_copy(
                        obuf, out_hbm.at[pl.ds(row0 + (g - 2) * R, R)], osem
                    ).wait()

                def blend_row(r, c2):
                    @plsc.parallel_loop(0, D // LANES, unroll=8)
                    def _blend_col(c):
                        xb = gbuf_b[r, pl.ds(c * LANES, LANES)]
                        xa = gbuf_a[r, pl.ds(c * LANES, LANES)]
                        obuf[r, pl.ds(c * LANES, LANES)] = xb + wv * (xa - xb)
                    return c2

                lax.fori_loop(0, R, blend_row, 0)
                pltpu.async_copy(obuf, out_hbm.at[pl.ds(row0 + g * R, R)], osem)
            return carry

        lax.fori_loop(0, n_chunks // 2, pair, 0)

        for par in range(2):
            obuf, osem = obufs[par]
            g_last = n_chunks - 2 + par
            pltpu.make_async_copy(
                obuf, out_hbm.at[pl.ds(row0 + g_last * R, R)], osem
            ).wait()

    return sc_kernel


def kernel(keyframe_attns, w, before_idx, after_idx, corr_before, corr_after):
    K, S, D = keyframe_attns.shape
    B, L = corr_before.shape
    nw = 32
    table = keyframe_attns.reshape(K * S, D)
    corr_b = corr_before.reshape(B * L).astype(jnp.int32)
    corr_a = corr_after.reshape(B * L).astype(jnp.int32)
    base_b = jnp.broadcast_to((before_idx.astype(jnp.int32) * S)[:, None], (B, LANES))
    base_a = jnp.broadcast_to((after_idx.astype(jnp.int32) * S)[:, None], (B, LANES))
    w_rep = jnp.broadcast_to(w.astype(jnp.float32)[:, None], (B, LANES))
    sc = _make_sc_kernel(K * S, D, B * L, B, L, nw)
    out = sc(table, corr_b, corr_a, base_b, base_a, w_rep)
    return out.reshape(B, L, D)


# speedup vs baseline: 4.0127x; 1.0250x over previous
import functools

import jax
import jax.numpy as jnp
from jax import lax
from jax.experimental import pallas as pl
from jax.experimental.pallas import tpu as pltpu
from jax.experimental.pallas import tpu_sc as plsc

LANES = 16


def _make_sc_kernel(KS, D, BL, B, L, nw, interpret=False):
    rpw = BL // nw
    R = 16
    NBUF = 4
    n_chunks = rpw // R

    mesh = plsc.VectorSubcoreMesh(core_axis_name="c", subcore_axis_name="s")
    NC = mesh.num_cores

    @functools.partial(
        pl.kernel,
        out_type=jax.ShapeDtypeStruct((BL, D), jnp.float32),
        mesh=mesh,
        scratch_types=[
            pltpu.VMEM((rpw,), jnp.int32),
            pltpu.VMEM((rpw,), jnp.int32),
            pltpu.VMEM((LANES,), jnp.int32),
            pltpu.VMEM((LANES,), jnp.int32),
            pltpu.VMEM((LANES,), jnp.float32),
            pltpu.VMEM((R, D), jnp.float32),
            pltpu.VMEM((R, D), jnp.float32),
            pltpu.VMEM((R, D), jnp.float32),
            pltpu.VMEM((R, D), jnp.float32),
            pltpu.VMEM((R, D), jnp.float32),
            pltpu.VMEM((R, D), jnp.float32),
            pltpu.VMEM((R, D), jnp.float32),
            pltpu.VMEM((R, D), jnp.float32),
            pltpu.VMEM((R, D), jnp.float32),
            pltpu.VMEM((R, D), jnp.float32),
            pltpu.SemaphoreType.DMA,
            pltpu.SemaphoreType.DMA,
            pltpu.SemaphoreType.DMA,
            pltpu.SemaphoreType.DMA,
            pltpu.SemaphoreType.DMA,
            pltpu.SemaphoreType.DMA,
            pltpu.SemaphoreType.DMA,
        ],
        interpret=interpret,
    )
    def sc_kernel(table_hbm, corr_b_hbm, corr_a_hbm, base_b_hbm, base_a_hbm,
                  w_hbm, out_hbm, idxb_v, idxa_v, bb_v, ba_v, w_v,
                  gb0, gb1, gb2, gb3, ga0, ga1, ga2, ga3, ob0, ob1,
                  gsem0, gsem1, gsem2, gsem3, osem0, osem1, psem):
        wid = lax.axis_index("s") * NC + lax.axis_index("c")
        row0 = wid * rpw
        b = row0 // L

        stage = (
            pltpu.async_copy(corr_b_hbm.at[pl.ds(row0, rpw)], idxb_v, psem),
            pltpu.async_copy(corr_a_hbm.at[pl.ds(row0, rpw)], idxa_v, psem),
            pltpu.async_copy(base_b_hbm.at[b], bb_v, psem),
            pltpu.async_copy(base_a_hbm.at[b], ba_v, psem),
            pltpu.async_copy(w_hbm.at[b], w_v, psem),
        )
        for cp in stage:
            cp.wait()
        bb = bb_v[...]
        ba = ba_v[...]
        wv = w_v[...]

        gbufs = ((gb0, ga0, gsem0), (gb1, ga1, gsem1),
                 (gb2, ga2, gsem2), (gb3, ga3, gsem3))
        obufs = ((ob0, osem0), (ob1, osem1))

        def gather_descs(g, par):
            gbuf_b, gbuf_a, gsem = gbufs[par]
            idx_b = idxb_v[pl.ds(g * R, R)] + bb
            idx_a = idxa_v[pl.ds(g * R, R)] + ba
            db = pltpu.make_async_copy(table_hbm.at[idx_b], gbuf_b, gsem)
            da = pltpu.make_async_copy(table_hbm.at[idx_a], gbuf_a, gsem)
            return db, da

        for g0 in range(NBUF - 1):
            dpb, dpa = gather_descs(g0, g0)
            dpb.start()
            dpa.start()

        def quad(go, carry):
            for par in range(NBUF):
                g = go * NBUF + par
                gbuf_b, gbuf_a, _ = gbufs[par]
                obuf, osem = obufs[par % 2]

                @pl.when(g + NBUF - 1 < n_chunks)
                def _issue_ahead():
                    dnb, dna = gather_descs(g + NBUF - 1, (par + NBUF - 1) % NBUF)
                    dnb.start()
                    dna.start()

                dwb, dwa = gather_descs(g, par)
                dwb.wait()
                dwa.wait()

                @pl.when(g >= 2)
                def _drain_prev_write():
                    pltpu.make_async_copy(
                        obuf, out_hbm.at[pl.ds(row0 + (g - 2) * R, R)], osem
                    ).wait()

                def blend_row(r, c2):
                    @plsc.parallel_loop(0, D // LANES, unroll=16)
                    def _blend_col(c):
                        xb = gbuf_b[r, pl.ds(c * LANES, LANES)]
                        xa = gbuf_a[r, pl.ds(c * LANES, LANES)]
                        obuf[r, pl.ds(c * LANES, LANES)] = xb + wv * (xa - xb)
                    return c2

                lax.fori_loop(0, R, blend_row, 0)
                pltpu.async_copy(obuf, out_hbm.at[pl.ds(row0 + g * R, R)], osem)
            return carry

        lax.fori_loop(0, n_chunks // NBUF, quad, 0)

        for j in range(2):
            g_last = n_chunks - 2 + j
            obuf, osem = obufs[g_last % 2]
            pltpu.make_async_copy(
                obuf, out_hbm.at[pl.ds(row0 + g_last * R, R)], osem
            ).wait()

    return sc_kernel


def kernel(keyframe_attns, w, before_idx, after_idx, corr_before, corr_after):
    K, S, D = keyframe_attns.shape
    B, L = corr_before.shape
    nw = 32
    table = keyframe_attns.reshape(K * S, D)
    corr_b = corr_before.reshape(B * L).astype(jnp.int32)
    corr_a = corr_after.reshape(B * L).astype(jnp.int32)
    base_b = jnp.broadcast_to((before_idx.astype(jnp.int32) * S)[:, None], (B, LANES))
    base_a = jnp.broadcast_to((after_idx.astype(jnp.int32) * S)[:, None], (B, LANES))
    w_rep = jnp.broadcast_to(w.astype(jnp.float32)[:, None], (B, LANES))
    sc = _make_sc_kernel(K * S, D, B * L, B, L, nw)
    out = sc(table, corr_b, corr_a, base_b, base_a, w_rep)
    return out.reshape(B, L, D)
